# baseline (speedup 1.0000x reference)
import jax
import jax.numpy as jnp
from jax.experimental import pallas as pl

N = 10000
G = 128


def _mlp_body(gr_ref, wm1_ref, bm1_ref, wm2_ref, bm2_ref, out_ref):
    h = jnp.maximum(gr_ref[...] @ wm1_ref[...] + bm1_ref[...][None, :], 0.0)
    out_ref[...] = h @ wm2_ref[...] + bm2_ref[...][None, :]


def kernel(x, edge_index, edge_attr, batch, W0, b0, W1, b1, W2, b2, We, Wm1, bm1, Wm2, bm2):
    src = edge_index[0]
    dst = edge_index[1]
    deg = jnp.bincount(dst, length=N).astype(jnp.float32)
    deg = jnp.clip(deg, 1.0)
    dinv = jax.lax.rsqrt(deg)
    norm = dinv[src] * dinv[dst]
    edge_emb = edge_attr @ We

    def layer(h, W, b):
        h2 = h @ W
        msg = h2[src] * norm[:, None] + edge_emb
        agg = jax.ops.segment_sum(msg, dst, num_segments=N)
        return agg + b

    h = jax.nn.relu(layer(x, W0, b0))
    h = jax.nn.relu(layer(h, W1, b1))
    h = jax.nn.relu(layer(h, W2, b2))
    graph_rep = jax.ops.segment_max(h, batch, num_segments=G)
    graph_rep = jnp.where(jnp.isfinite(graph_rep), graph_rep, 0.0)
    pred = pl.pallas_call(
        _mlp_body,
        out_shape=jax.ShapeDtypeStruct((G, Wm2.shape[1]), jnp.float32),
    )(graph_rep, Wm1, bm1, Wm2, bm2)
    return pred


# speedup vs baseline: 6.6382x; 6.6382x over previous
import functools

import jax
import jax.numpy as jnp
from jax import lax
from jax.experimental import pallas as pl
from jax.experimental.pallas import tpu as pltpu
from jax.experimental.pallas import tpu_sc as plsc

N = 10000
E = 320000
C = 128
DE = 16
G = 128
NC = 2
NS = 16
NW = NC * NS
EW = E // NW
K = 80
NCHUNK = EW // K
RPW = 624
RTAIL = N - NS * RPW

_mesh = plsc.VectorSubcoreMesh(core_axis_name="c", subcore_axis_name="s")


def _wid(c, s):
    return c * NS + s


def _zero_slab(z_hbm, sh, s):
    r0 = s * RPW
    pltpu.sync_copy(z_hbm, sh.at[pl.ds(r0, RPW)])

    @pl.when(s == NS - 1)
    def _():
        pltpu.sync_copy(z_hbm.at[pl.ds(0, RTAIL)],
                        sh.at[pl.ds(NS * RPW, RTAIL)])


def _write_slab(sh, out, s, base):
    r0 = s * RPW
    pltpu.sync_copy(sh.at[pl.ds(r0, RPW)], out.at[pl.ds(base + r0, RPW)])

    @pl.when(s == NS - 1)
    def _():
        pltpu.sync_copy(sh.at[pl.ds(NS * RPW, RTAIL)],
                        out.at[pl.ds(base + NS * RPW, RTAIL)])


def _pass_a_body(eaw_hbm, dst_hbm, z625_hbm, comb,
                 idst, rows, comb_sh):
    c = lax.axis_index("c")
    s = lax.axis_index("s")
    w = _wid(c, s)
    _zero_slab(z625_hbm, comb_sh, s)
    plsc.subcore_barrier()

    def body(j, carry):
        off = w * EW + j * K
        pltpu.sync_copy(dst_hbm.at[pl.ds(off, K)], idst)
        pltpu.sync_copy(eaw_hbm.at[pl.ds(off, K)], rows)
        pltpu.sync_copy(rows, comb_sh.at[idst], add=True)
        return carry

    lax.fori_loop(0, NCHUNK, body, 0)
    plsc.subcore_barrier()
    _write_slab(comb_sh, comb, s, c * N)


_pass_a = pl.kernel(
    _pass_a_body,
    out_type=jax.ShapeDtypeStruct((2 * N, C), jnp.float32),
    mesh=_mesh,
    scratch_types=[
        pltpu.VMEM((K,), jnp.int32),
        pltpu.VMEM((K, C), jnp.float32),
        pltpu.VMEM_SHARED((N, C), jnp.float32),
    ],
)


def _pass_b_body(g_hbm, src_hbm, dst_hbm, z625_hbm, sp,
                 isrc, idst, rows, s_sh, gsem):
    c = lax.axis_index("c")
    s = lax.axis_index("s")
    w = _wid(c, s)
    _zero_slab(z625_hbm, s_sh, s)
    plsc.subcore_barrier()

    def body(j, carry):
        off = w * EW + j * K
        pltpu.sync_copy(src_hbm.at[pl.ds(off, K)], isrc)
        pltpu.sync_copy(dst_hbm.at[pl.ds(off, K)], idst)
        pltpu.async_copy(g_hbm.at[isrc], rows, gsem).wait()
        pltpu.sync_copy(rows, s_sh.at[idst], add=True)
        return carry

    lax.fori_loop(0, NCHUNK, body, 0)
    plsc.subcore_barrier()
    _write_slab(s_sh, sp, s, c * N)


_pass_b = pl.kernel(
    _pass_b_body,
    out_type=jax.ShapeDtypeStruct((2 * N, C), jnp.float32),
    mesh=_mesh,
    scratch_types=[
        pltpu.VMEM((K,), jnp.int32),
        pltpu.VMEM((K,), jnp.int32),
        pltpu.VMEM((K, C), jnp.float32),
        pltpu.VMEM_SHARED((N, C), jnp.float32),
        pltpu.SemaphoreType.DMA,
    ],
)


RB = 400
NB = N // RB


def _dinv_of(d0_ref, d1_ref):
    deg = d0_ref[...][:, 0:1] + d1_ref[...][:, 0:1]
    return lax.rsqrt(jnp.maximum(deg, 1.0))


def _tc1_body(x_ref, w0_ref, d0_ref, d1_ref, out_ref):
    dinv = _dinv_of(d0_ref, d1_ref)
    out_ref[...] = jnp.dot(x_ref[...], w0_ref[...],
                           preferred_element_type=jnp.float32) * dinv


def _tc_mid_body(s0_ref, s1_ref, e0_ref, e1_ref, d0_ref, d1_ref,
                 we_ref, b_ref, w_ref, out_ref):
    dinv = _dinv_of(d0_ref, d1_ref)
    esum = jnp.dot(e0_ref[...] + e1_ref[...], we_ref[...],
                   preferred_element_type=jnp.float32)
    h = jnp.maximum((s0_ref[...] + s1_ref[...]) * dinv + esum + b_ref[...], 0.0)
    out_ref[...] = jnp.dot(h, w_ref[...],
                           preferred_element_type=jnp.float32) * dinv


def _tc_pool_body(s0_ref, s1_ref, e0_ref, e1_ref, d0_ref, d1_ref,
                  we_ref, b_ref, bsm_ref, bvec_ref, out_ref):
    i = pl.program_id(0)
    dinv = _dinv_of(d0_ref, d1_ref)
    esum = jnp.dot(e0_ref[...] + e1_ref[...], we_ref[...],
                   preferred_element_type=jnp.float32)
    h = jnp.maximum(
        (s0_ref[...] + s1_ref[...]) * dinv + esum + b_ref[...], 0.0)

    @pl.when(i == 0)
    def _():
        out_ref[...] = jnp.zeros_like(out_ref)

    bvec = bvec_ref[...]
    gmin = bsm_ref[0, 0, 0]
    gmax = bsm_ref[0, 0, RB - 1]

    def upd(g, carry):
        colmax = jnp.max(jnp.where(bvec == g, h, 0.0), axis=0, keepdims=True)
        out_ref[pl.ds(g, 1), :] = jnp.maximum(out_ref[pl.ds(g, 1), :], colmax)
        return carry

    lax.fori_loop(gmin, gmax + 1, upd, 0)


def _tc_mlp_body(gr_ref, wm1_ref, bm1_ref, wm2_ref, bm2_ref, out_ref):
    h = jnp.maximum(jnp.dot(gr_ref[...], wm1_ref[...],
                            preferred_element_type=jnp.float32)
                    + bm1_ref[...], 0.0)
    out_ref[...] = jnp.dot(h, wm2_ref[...],
                           preferred_element_type=jnp.float32) + bm2_ref[...]


def _row_spec(cols):
    return pl.BlockSpec((RB, cols), lambda i: (i, 0))


def _full_spec(rows, cols):
    return pl.BlockSpec((rows, cols), lambda i: (0, 0))


_tc1 = pl.pallas_call(
    _tc1_body,
    grid=(NB,),
    in_specs=[_row_spec(C), _full_spec(C, C), _row_spec(DE), _row_spec(DE)],
    out_specs=_row_spec(C),
    out_shape=jax.ShapeDtypeStruct((N, C), jnp.float32),
)

_tc_mid = pl.pallas_call(
    _tc_mid_body,
    grid=(NB,),
    in_specs=[_row_spec(C), _row_spec(C), _row_spec(DE), _row_spec(DE),
              _row_spec(DE), _row_spec(DE), _full_spec(DE, C),
              _full_spec(1, C), _full_spec(C, C)],
    out_specs=_row_spec(C),
    out_shape=jax.ShapeDtypeStruct((N, C), jnp.float32),
)

_tc_pool = pl.pallas_call(
    _tc_pool_body,
    grid=(NB,),
    in_specs=[_row_spec(C), _row_spec(C), _row_spec(DE), _row_spec(DE),
              _row_spec(DE), _row_spec(DE), _full_spec(DE, C),
              _full_spec(1, C),
              pl.BlockSpec((1, 1, RB), lambda i: (i, 0, 0),
                           memory_space=pltpu.SMEM),
              pl.BlockSpec((RB, 1), lambda i: (i, 0))],
    out_specs=pl.BlockSpec((G, C), lambda i: (0, 0)),
    out_shape=jax.ShapeDtypeStruct((G, C), jnp.float32),
)


def _mlp_call(gr, Wm1, bm1, Wm2, bm2):
    return pl.pallas_call(
        _tc_mlp_body,
        out_shape=jax.ShapeDtypeStruct((G, Wm2.shape[1]), jnp.float32),
    )(gr, Wm1, bm1.reshape(1, -1), Wm2, bm2.reshape(1, -1))


def kernel(x, edge_index, edge_attr, batch, W0, b0, W1, b1, W2, b2,
           We, Wm1, bm1, Wm2, bm2):
    src = edge_index[0]
    dst = edge_index[1]
    z625 = jnp.zeros((RPW, C), jnp.float32)

    eaw = jnp.concatenate(
        [edge_attr, jnp.ones((E, 1), jnp.float32),
         jnp.zeros((E, C - DE - 1), jnp.float32)], axis=1)
    comb = _pass_a(eaw, dst, z625)
    d0, d1 = comb[:N, DE:2 * DE], comb[N:, DE:2 * DE]
    e0, e1 = comb[:N, :DE], comb[N:, :DE]

    def _sb(g):
        return _pass_b(g, src, dst, z625)

    g = _tc1(x, W0, d0, d1)
    sp = _sb(g)
    g = _tc_mid(sp[:N], sp[N:], e0, e1, d0, d1, We, b0.reshape(1, -1), W1)
    sp = _sb(g)
    g = _tc_mid(sp[:N], sp[N:], e0, e1, d0, d1, We, b1.reshape(1, -1), W2)
    sp = _sb(g)
    gr = _tc_pool(sp[:N], sp[N:], e0, e1, d0, d1, We, b2.reshape(1, -1),
                  batch.reshape(NB, 1, RB), batch.reshape(N, 1))
    return _mlp_call(gr, Wm1, bm1, Wm2, bm2)


# speedup vs baseline: 10.7678x; 1.6221x over previous
import functools

import jax
import jax.numpy as jnp
from jax import lax
from jax.experimental import pallas as pl
from jax.experimental.pallas import tpu as pltpu
from jax.experimental.pallas import tpu_sc as plsc

N = 10000
E = 320000
C = 128
DE = 16
G = 128
NC = 2
NS = 16
NW = NC * NS
EW = E // NW
K = 80
NCHUNK = EW // K
RPW = 624
RTAIL = N - NS * RPW

_mesh = plsc.VectorSubcoreMesh(core_axis_name="c", subcore_axis_name="s")


def _wid(c, s):
    return c * NS + s


def _zero_slab(z_hbm, sh, s):
    r0 = s * RPW
    pltpu.sync_copy(z_hbm, sh.at[pl.ds(r0, RPW)])

    @pl.when(s == NS - 1)
    def _():
        pltpu.sync_copy(z_hbm.at[pl.ds(0, RTAIL)],
                        sh.at[pl.ds(NS * RPW, RTAIL)])


def _write_slab(sh, out, s, base):
    r0 = s * RPW
    pltpu.sync_copy(sh.at[pl.ds(r0, RPW)], out.at[pl.ds(base + r0, RPW)])

    @pl.when(s == NS - 1)
    def _():
        pltpu.sync_copy(sh.at[pl.ds(NS * RPW, RTAIL)],
                        out.at[pl.ds(base + NS * RPW, RTAIL)])


def _pass_a_body(eaw_hbm, dst3_hbm, z625_hbm, comb,
                 dall, rows0, rows1, comb_sh, rs0, rs1, ss0, ss1):
    c = lax.axis_index("c")
    s = lax.axis_index("s")
    w = _wid(c, s)
    _zero_slab(z625_hbm, comb_sh, s)
    pltpu.sync_copy(dst3_hbm.at[w], dall)
    plsc.subcore_barrier()
    base = w * EW

    def rd(j, rows, sem):
        pltpu.async_copy(eaw_hbm.at[pl.ds(base + j * K, K)], rows, sem)

    def rd_wait(rows, sem):
        pltpu.make_async_copy(eaw_hbm.at[pl.ds(base, K)], rows, sem).wait()

    def sc(j, rows, sem):
        pltpu.async_copy(rows, comb_sh.at[dall.at[j]], sem, add=True)

    def sc_wait(rows, sem):
        pltpu.make_async_copy(rows, comb_sh.at[dall.at[0]], sem).wait()

    rd(0, rows0, rs0)
    rd(1, rows1, rs1)

    def body(t, carry):
        j0 = 2 * t
        rd_wait(rows0, rs0)
        sc(j0, rows0, ss0)
        rd_wait(rows1, rs1)
        sc(j0 + 1, rows1, ss1)
        sc_wait(rows0, ss0)

        @pl.when(j0 + 2 < NCHUNK)
        def _():
            rd(j0 + 2, rows0, rs0)

        sc_wait(rows1, ss1)

        @pl.when(j0 + 3 < NCHUNK)
        def _():
            rd(j0 + 3, rows1, rs1)

        return carry

    lax.fori_loop(0, NCHUNK // 2, body, 0)
    rd_wait(rows0, rs0)
    sc(NCHUNK - 1, rows0, ss0)
    sc_wait(rows0, ss0)
    plsc.subcore_barrier()
    _write_slab(comb_sh, comb, s, c * N)


_pass_a = pl.kernel(
    _pass_a_body,
    out_type=jax.ShapeDtypeStruct((2 * N, C), jnp.float32),
    mesh=_mesh,
    scratch_types=[
        pltpu.VMEM((NCHUNK, K), jnp.int32),
        pltpu.VMEM((K, C), jnp.float32),
        pltpu.VMEM((K, C), jnp.float32),
        pltpu.VMEM_SHARED((N, C), jnp.float32),
        pltpu.SemaphoreType.DMA,
        pltpu.SemaphoreType.DMA,
        pltpu.SemaphoreType.DMA,
        pltpu.SemaphoreType.DMA,
    ],
)


def _pass_b_body(g_hbm, src_hbm, dst3_hbm, z625_hbm, sp,
                 sall, dall, rows0, rows1, s_sh, gs0, gs1, ss0, ss1):
    c = lax.axis_index("c")
    s = lax.axis_index("s")
    w = _wid(c, s)
    _zero_slab(z625_hbm, s_sh, s)
    base = w * EW
    pltpu.sync_copy(src_hbm.at[pl.ds(base, EW)], sall)
    pltpu.sync_copy(dst3_hbm.at[w], dall)
    plsc.subcore_barrier()

    def ga(j, rows, sem):
        pltpu.async_copy(g_hbm.at[sall.at[pl.ds(j * K, K)]], rows, sem)

    def ga_wait(rows, sem):
        pltpu.make_async_copy(g_hbm.at[sall.at[pl.ds(0, K)]], rows,
                              sem).wait()

    def sc(j, rows, sem):
        pltpu.async_copy(rows, s_sh.at[dall.at[j]], sem, add=True)

    def sc_wait(rows, sem):
        pltpu.make_async_copy(rows, s_sh.at[dall.at[0]], sem).wait()

    ga(0, rows0, gs0)
    ga(1, rows1, gs1)

    def body(t, carry):
        j0 = 2 * t
        ga_wait(rows0, gs0)
        sc(j0, rows0, ss0)
        ga_wait(rows1, gs1)
        sc(j0 + 1, rows1, ss1)
        sc_wait(rows0, ss0)

        @pl.when(j0 + 2 < NCHUNK)
        def _():
            ga(j0 + 2, rows0, gs0)

        sc_wait(rows1, ss1)

        @pl.when(j0 + 3 < NCHUNK)
        def _():
            ga(j0 + 3, rows1, gs1)

        return carry

    lax.fori_loop(0, NCHUNK // 2, body, 0)
    ga_wait(rows0, gs0)
    sc(NCHUNK - 1, rows0, ss0)
    sc_wait(rows0, ss0)
    plsc.subcore_barrier()
    _write_slab(s_sh, sp, s, c * N)


_pass_b = pl.kernel(
    _pass_b_body,
    out_type=jax.ShapeDtypeStruct((2 * N, C), jnp.float32),
    mesh=_mesh,
    scratch_types=[
        pltpu.VMEM((EW,), jnp.int32),
        pltpu.VMEM((NCHUNK, K), jnp.int32),
        pltpu.VMEM((K, C), jnp.float32),
        pltpu.VMEM((K, C), jnp.float32),
        pltpu.VMEM_SHARED((N, C), jnp.float32),
        pltpu.SemaphoreType.DMA,
        pltpu.SemaphoreType.DMA,
        pltpu.SemaphoreType.DMA,
        pltpu.SemaphoreType.DMA,
    ],
)


RB = 400
NB = N // RB


def _dinv_of(d0_ref, d1_ref):
    deg = d0_ref[...][:, 0:1] + d1_ref[...][:, 0:1]
    return lax.rsqrt(jnp.maximum(deg, 1.0))


def _tc1_body(x_ref, w0_ref, d0_ref, d1_ref, out_ref):
    dinv = _dinv_of(d0_ref, d1_ref)
    out_ref[...] = jnp.dot(x_ref[...], w0_ref[...],
                           preferred_element_type=jnp.float32) * dinv


def _tc_mid_body(s0_ref, s1_ref, e0_ref, e1_ref, d0_ref, d1_ref,
                 we_ref, b_ref, w_ref, out_ref):
    dinv = _dinv_of(d0_ref, d1_ref)
    esum = jnp.dot(e0_ref[...] + e1_ref[...], we_ref[...],
                   preferred_element_type=jnp.float32)
    h = jnp.maximum((s0_ref[...] + s1_ref[...]) * dinv + esum + b_ref[...], 0.0)
    out_ref[...] = jnp.dot(h, w_ref[...],
                           preferred_element_type=jnp.float32) * dinv


def _tc_pool_body(s0_ref, s1_ref, e0_ref, e1_ref, d0_ref, d1_ref,
                  we_ref, b_ref, bsm_ref, bvec_ref, out_ref):
    i = pl.program_id(0)
    dinv = _dinv_of(d0_ref, d1_ref)
    esum = jnp.dot(e0_ref[...] + e1_ref[...], we_ref[...],
                   preferred_element_type=jnp.float32)
    h = jnp.maximum(
        (s0_ref[...] + s1_ref[...]) * dinv + esum + b_ref[...], 0.0)

    @pl.when(i == 0)
    def _():
        out_ref[...] = jnp.zeros_like(out_ref)

    bvec = bvec_ref[...]
    gmin = bsm_ref[0, 0, 0]
    gmax = bsm_ref[0, 0, RB - 1]

    def upd(g, carry):
        colmax = jnp.max(jnp.where(bvec == g, h, 0.0), axis=0, keepdims=True)
        out_ref[pl.ds(g, 1), :] = jnp.maximum(out_ref[pl.ds(g, 1), :], colmax)
        return carry

    lax.fori_loop(gmin, gmax + 1, upd, 0)


def _tc_mlp_body(gr_ref, wm1_ref, bm1_ref, wm2_ref, bm2_ref, out_ref):
    h = jnp.maximum(jnp.dot(gr_ref[...], wm1_ref[...],
                            preferred_element_type=jnp.float32)
                    + bm1_ref[...], 0.0)
    out_ref[...] = jnp.dot(h, wm2_ref[...],
                           preferred_element_type=jnp.float32) + bm2_ref[...]


def _row_spec(cols):
    return pl.BlockSpec((RB, cols), lambda i: (i, 0))


def _full_spec(rows, cols):
    return pl.BlockSpec((rows, cols), lambda i: (0, 0))


_tc1 = pl.pallas_call(
    _tc1_body,
    grid=(NB,),
    in_specs=[_row_spec(C), _full_spec(C, C), _row_spec(DE), _row_spec(DE)],
    out_specs=_row_spec(C),
    out_shape=jax.ShapeDtypeStruct((N, C), jnp.float32),
)

_tc_mid = pl.pallas_call(
    _tc_mid_body,
    grid=(NB,),
    in_specs=[_row_spec(C), _row_spec(C), _row_spec(DE), _row_spec(DE),
              _row_spec(DE), _row_spec(DE), _full_spec(DE, C),
              _full_spec(1, C), _full_spec(C, C)],
    out_specs=_row_spec(C),
    out_shape=jax.ShapeDtypeStruct((N, C), jnp.float32),
)

_tc_pool = pl.pallas_call(
    _tc_pool_body,
    grid=(NB,),
    in_specs=[_row_spec(C), _row_spec(C), _row_spec(DE), _row_spec(DE),
              _row_spec(DE), _row_spec(DE), _full_spec(DE, C),
              _full_spec(1, C),
              pl.BlockSpec((1, 1, RB), lambda i: (i, 0, 0),
                           memory_space=pltpu.SMEM),
              pl.BlockSpec((RB, 1), lambda i: (i, 0))],
    out_specs=pl.BlockSpec((G, C), lambda i: (0, 0)),
    out_shape=jax.ShapeDtypeStruct((G, C), jnp.float32),
)


def _mlp_call(gr, Wm1, bm1, Wm2, bm2):
    return pl.pallas_call(
        _tc_mlp_body,
        out_shape=jax.ShapeDtypeStruct((G, Wm2.shape[1]), jnp.float32),
    )(gr, Wm1, bm1.reshape(1, -1), Wm2, bm2.reshape(1, -1))


def kernel(x, edge_index, edge_attr, batch, W0, b0, W1, b1, W2, b2,
           We, Wm1, bm1, Wm2, bm2):
    src = edge_index[0]
    dst = edge_index[1]
    z625 = jnp.zeros((RPW, C), jnp.float32)

    eaw = jnp.concatenate(
        [edge_attr, jnp.ones((E, 1), jnp.float32),
         jnp.zeros((E, C - DE - 1), jnp.float32)], axis=1)
    dst3 = dst.reshape(NW, NCHUNK, K)
    comb = _pass_a(eaw, dst3, z625)
    d0, d1 = comb[:N, DE:2 * DE], comb[N:, DE:2 * DE]
    e0, e1 = comb[:N, :DE], comb[N:, :DE]

    def _sb(g):
        return _pass_b(g, src, dst3, z625)

    g = _tc1(x, W0, d0, d1)
    sp = _sb(g)
    g = _tc_mid(sp[:N], sp[N:], e0, e1, d0, d1, We, b0.reshape(1, -1), W1)
    sp = _sb(g)
    g = _tc_mid(sp[:N], sp[N:], e0, e1, d0, d1, We, b1.reshape(1, -1), W2)
    sp = _sb(g)
    gr = _tc_pool(sp[:N], sp[N:], e0, e1, d0, d1, We, b2.reshape(1, -1),
                  batch.reshape(NB, 1, RB), batch.reshape(N, 1))
    return _mlp_call(gr, Wm1, bm1, Wm2, bm2)
